# baseline (speedup 1.0000x reference)
import functools

import jax
import jax.numpy as jnp
from jax import lax
from jax.experimental import pallas as pl
from jax.experimental.pallas import tpu as pltpu
from jax.experimental.pallas import tpu_sc as plsc

BATCH = 1024
ROW = 1024
PAIRS = 512
NCLS = 7

_NC = 2
_NS = 16
_NW = _NC * _NS
_ROWS_PER_W = BATCH // _NW
_CHUNK = _ROWS_PER_W * ROW
_VECS = _CHUNK // 16


def _sc_gather_body(x_hbm, idx_hbm, out_hbm, xv, iv, ov):
    wid = lax.axis_index("s") * _NC + lax.axis_index("c")
    r0 = wid * _ROWS_PER_W
    pltpu.sync_copy(x_hbm.at[pl.ds(r0, _ROWS_PER_W)], xv)
    pltpu.sync_copy(idx_hbm.at[pl.ds(r0, _ROWS_PER_W)], iv)

    def row_body(r, carry):
        rv = jnp.full((16,), r, dtype=jnp.int32)
        for t in range(ROW // 16):
            sl = pl.ds(t * 16, 16)
            ov[r, sl] = plsc.load_gather(xv, [rv, iv[r, sl]])
        return carry

    lax.fori_loop(0, _ROWS_PER_W, row_body, 0)
    pltpu.sync_copy(ov, out_hbm.at[pl.ds(r0, _ROWS_PER_W)])


@functools.cache
def _sc_gather():
    return pl.kernel(
        _sc_gather_body,
        mesh=plsc.VectorSubcoreMesh(
            core_axis_name="c", subcore_axis_name="s",
            num_cores=_NC, num_subcores=_NS),
        out_type=jax.ShapeDtypeStruct((BATCH, ROW), jnp.float32),
        compiler_params=pltpu.CompilerParams(needs_layout_passes=False),
        scratch_types=[
            pltpu.VMEM((_ROWS_PER_W, ROW), jnp.float32),
            pltpu.VMEM((_ROWS_PER_W, ROW), jnp.int32),
            pltpu.VMEM((_ROWS_PER_W, ROW), jnp.float32),
        ],
    )


_S = 128


def _tc_body(gv_ref, a2_ref, b2_ref, o_ref):
    px0 = gv_ref[:, :PAIRS]
    px1 = gv_ref[:, PAIRS:]
    c0 = jnp.cos(px0)
    cs = jnp.cos(px0 + px1)
    cd = jnp.cos(px0 - px1)
    tb = cd - cs
    o1v = 0.5 + 0.25 * (cd + cs)
    o1_4 = o1v.reshape(_S, 4, 128)
    for i in range(NCLS):
        a = a2_ref[i][None, :]
        b = b2_ref[i][None, :]
        o0c = 0.5 + a * c0 - b * tb
        o0_4 = o0c.reshape(_S, 4, 128)
        o_ref[:, i, :, :] = jnp.stack([o0_4, o1_4], axis=2).reshape(_S, 8, 128)


def kernel(x, rz_params, ry_params, fixed_pair_indices, random_pair_indices):
    B = x.shape[0]
    f0 = fixed_pair_indices[:, 0].astype(jnp.int32)
    f1 = fixed_pair_indices[:, 1].astype(jnp.int32)
    r0 = random_pair_indices[:, :, 0].astype(jnp.int32)
    r1 = random_pair_indices[:, :, 1].astype(jnp.int32)
    f0b = jnp.broadcast_to(f0[None, :], (B, f0.shape[0]))
    f1b = jnp.broadcast_to(f1[None, :], (B, f1.shape[0]))
    gidx = jnp.concatenate([f0b, r0, f1b, r1], axis=1)

    gv = _sc_gather()(x.reshape(B, ROW), gidx)

    cf = 0.5 * jnp.cos(ry_params)
    sfct = 0.25 * jnp.sin(ry_params) * jnp.cos(rz_params)
    a2 = jnp.broadcast_to(cf[:, None], (NCLS, PAIRS)).astype(jnp.float32)
    b2 = jnp.broadcast_to(sfct[:, None], (NCLS, PAIRS)).astype(jnp.float32)

    o4 = pl.pallas_call(
        _tc_body,
        grid=(B // _S,),
        in_specs=[
            pl.BlockSpec((_S, ROW), lambda i: (i, 0)),
            pl.BlockSpec((NCLS, PAIRS), lambda i: (0, 0)),
            pl.BlockSpec((NCLS, PAIRS), lambda i: (0, 0)),
        ],
        out_specs=pl.BlockSpec((_S, NCLS, 8, 128), lambda i: (i, 0, 0, 0)),
        out_shape=jax.ShapeDtypeStruct((B, NCLS, 8, 128), jnp.float32),
    )(gv, a2, b2)

    out = o4.reshape(B, NCLS, 4, 2, 128)
    out = jnp.transpose(out, (0, 1, 2, 4, 3))
    return out.reshape(B, NCLS, PAIRS, 2)


# speedup vs baseline: 1.0071x; 1.0071x over previous
import functools

import jax
import jax.numpy as jnp
from jax import lax
from jax.experimental import pallas as pl
from jax.experimental.pallas import tpu as pltpu
from jax.experimental.pallas import tpu_sc as plsc

BATCH = 1024
ROW = 1024
PAIRS = 512
NCLS = 7

_NC = 2
_NS = 16
_NW = _NC * _NS
_ROWS_PER_W = BATCH // _NW
_CHUNK = _ROWS_PER_W * ROW
_VECS = _CHUNK // 16


def _sc_gather_body(x_hbm, idx_hbm, out_hbm, xv, iv, ov):
    wid = lax.axis_index("s") * _NC + lax.axis_index("c")
    r0 = wid * _ROWS_PER_W
    pltpu.sync_copy(x_hbm.at[pl.ds(r0, _ROWS_PER_W)], xv)
    pltpu.sync_copy(idx_hbm.at[pl.ds(r0, _ROWS_PER_W)], iv)

    def row_body(r, carry):
        rv = jnp.full((16,), r, dtype=jnp.int32)
        for t in range(ROW // 16):
            sl = pl.ds(t * 16, 16)
            ov[r, sl] = plsc.load_gather(xv, [rv, iv[r, sl]])
        return carry

    lax.fori_loop(0, _ROWS_PER_W, row_body, 0)
    pltpu.sync_copy(ov, out_hbm.at[pl.ds(r0, _ROWS_PER_W)])


@functools.cache
def _sc_gather():
    return pl.kernel(
        _sc_gather_body,
        mesh=plsc.VectorSubcoreMesh(
            core_axis_name="c", subcore_axis_name="s",
            num_cores=_NC, num_subcores=_NS),
        out_type=jax.ShapeDtypeStruct((BATCH, ROW), jnp.float32),
        compiler_params=pltpu.CompilerParams(needs_layout_passes=False),
        scratch_types=[
            pltpu.VMEM((_ROWS_PER_W, ROW), jnp.float32),
            pltpu.VMEM((_ROWS_PER_W, ROW), jnp.int32),
            pltpu.VMEM((_ROWS_PER_W, ROW), jnp.float32),
        ],
    )


_S = 64


def _tc_body(gv_ref, a2_ref, b2_ref, o_ref):
    px0 = gv_ref[:, :PAIRS]
    px1 = gv_ref[:, PAIRS:]
    c0 = jnp.cos(px0)
    cs = jnp.cos(px0 + px1)
    cd = jnp.cos(px0 - px1)
    tb = cd - cs
    o1v = 0.5 + 0.25 * (cd + cs)
    o1_4 = o1v.reshape(_S, 4, 128)
    for i in range(NCLS):
        a = a2_ref[i][None, :]
        b = b2_ref[i][None, :]
        o0c = 0.5 + a * c0 - b * tb
        o0_4 = o0c.reshape(_S, 4, 128)
        o_ref[:, i, :, :] = jnp.stack([o0_4, o1_4], axis=2).reshape(_S, 8, 128)


def kernel(x, rz_params, ry_params, fixed_pair_indices, random_pair_indices):
    B = x.shape[0]
    f0 = fixed_pair_indices[:, 0].astype(jnp.int32)
    f1 = fixed_pair_indices[:, 1].astype(jnp.int32)
    r0 = random_pair_indices[:, :, 0].astype(jnp.int32)
    r1 = random_pair_indices[:, :, 1].astype(jnp.int32)
    f0b = jnp.broadcast_to(f0[None, :], (B, f0.shape[0]))
    f1b = jnp.broadcast_to(f1[None, :], (B, f1.shape[0]))
    gidx = jnp.concatenate([f0b, r0, f1b, r1], axis=1)

    gv = _sc_gather()(x.reshape(B, ROW), gidx)

    cf = 0.5 * jnp.cos(ry_params)
    sfct = 0.25 * jnp.sin(ry_params) * jnp.cos(rz_params)
    a2 = jnp.broadcast_to(cf[:, None], (NCLS, PAIRS)).astype(jnp.float32)
    b2 = jnp.broadcast_to(sfct[:, None], (NCLS, PAIRS)).astype(jnp.float32)

    o4 = pl.pallas_call(
        _tc_body,
        grid=(B // _S,),
        in_specs=[
            pl.BlockSpec((_S, ROW), lambda i: (i, 0)),
            pl.BlockSpec((NCLS, PAIRS), lambda i: (0, 0)),
            pl.BlockSpec((NCLS, PAIRS), lambda i: (0, 0)),
        ],
        out_specs=pl.BlockSpec((_S, NCLS, 8, 128), lambda i: (i, 0, 0, 0)),
        out_shape=jax.ShapeDtypeStruct((B, NCLS, 8, 128), jnp.float32),
    )(gv, a2, b2)

    out = o4.reshape(B, NCLS, 4, 2, 128)
    out = jnp.transpose(out, (0, 1, 2, 4, 3))
    return out.reshape(B, NCLS, PAIRS, 2)


# speedup vs baseline: 1.0679x; 1.0604x over previous
import functools

import jax
import jax.numpy as jnp
from jax import lax
from jax.experimental import pallas as pl
from jax.experimental.pallas import tpu as pltpu
from jax.experimental.pallas import tpu_sc as plsc

BATCH = 1024
ROW = 1024
PAIRS = 512
NCLS = 7

_NC = 2
_NS = 16
_NW = _NC * _NS
_ROWS_PER_W = BATCH // _NW
_CHUNK = _ROWS_PER_W * ROW
_VECS = _CHUNK // 16


def _sc_gather_body(x_hbm, idx_hbm, out_hbm, xv, iv, ov):
    wid = lax.axis_index("s") * _NC + lax.axis_index("c")
    r0 = wid * _ROWS_PER_W
    pltpu.sync_copy(x_hbm.at[pl.ds(r0, _ROWS_PER_W)], xv)
    pltpu.sync_copy(idx_hbm.at[pl.ds(r0, _ROWS_PER_W)], iv)

    def row_body(r, carry):
        rv = jnp.full((16,), r, dtype=jnp.int32)
        for t in range(ROW // 16):
            sl = pl.ds(t * 16, 16)
            ov[r, sl] = plsc.load_gather(xv, [rv, iv[r, sl]])
        return carry

    lax.fori_loop(0, _ROWS_PER_W, row_body, 0)
    pltpu.sync_copy(ov, out_hbm.at[pl.ds(r0, _ROWS_PER_W)])


@functools.cache
def _sc_gather():
    return pl.kernel(
        _sc_gather_body,
        mesh=plsc.VectorSubcoreMesh(
            core_axis_name="c", subcore_axis_name="s",
            num_cores=_NC, num_subcores=_NS),
        out_type=jax.ShapeDtypeStruct((BATCH, ROW), jnp.float32),
        compiler_params=pltpu.CompilerParams(needs_layout_passes=False),
        scratch_types=[
            pltpu.VMEM((_ROWS_PER_W, ROW), jnp.float32),
            pltpu.VMEM((_ROWS_PER_W, ROW), jnp.int32),
            pltpu.VMEM((_ROWS_PER_W, ROW), jnp.float32),
        ],
    )


_S = 64


def _tc_body(gv_ref, a2_ref, b2_ref, o_ref):
    px0 = gv_ref[:, :PAIRS]
    px1 = gv_ref[:, PAIRS:]
    c0 = jnp.cos(px0)
    s0 = jnp.sin(px0)
    c1 = jnp.cos(px1)
    s1 = jnp.sin(px1)
    tb = s0 * s1
    o1v = 0.5 + 0.5 * (c0 * c1)
    o1_4 = o1v.reshape(_S, 4, 128)
    for i in range(NCLS):
        a = a2_ref[i][None, :]
        b = b2_ref[i][None, :]
        o0c = 0.5 + a * c0 - b * tb
        o0_4 = o0c.reshape(_S, 4, 128)
        o_ref[:, i, :, :] = jnp.stack([o0_4, o1_4], axis=2).reshape(_S, 8, 128)


def kernel(x, rz_params, ry_params, fixed_pair_indices, random_pair_indices):
    B = x.shape[0]
    f0 = fixed_pair_indices[:, 0].astype(jnp.int32)
    f1 = fixed_pair_indices[:, 1].astype(jnp.int32)
    r0 = random_pair_indices[:, :, 0].astype(jnp.int32)
    r1 = random_pair_indices[:, :, 1].astype(jnp.int32)
    f0b = jnp.broadcast_to(f0[None, :], (B, f0.shape[0]))
    f1b = jnp.broadcast_to(f1[None, :], (B, f1.shape[0]))
    gidx = jnp.concatenate([f0b, r0, f1b, r1], axis=1)

    gv = _sc_gather()(x.reshape(B, ROW), gidx)

    cf = 0.5 * jnp.cos(ry_params)
    sfct = 0.5 * jnp.sin(ry_params) * jnp.cos(rz_params)
    a2 = jnp.broadcast_to(cf[:, None], (NCLS, PAIRS)).astype(jnp.float32)
    b2 = jnp.broadcast_to(sfct[:, None], (NCLS, PAIRS)).astype(jnp.float32)

    o4 = pl.pallas_call(
        _tc_body,
        grid=(B // _S,),
        in_specs=[
            pl.BlockSpec((_S, ROW), lambda i: (i, 0)),
            pl.BlockSpec((NCLS, PAIRS), lambda i: (0, 0)),
            pl.BlockSpec((NCLS, PAIRS), lambda i: (0, 0)),
        ],
        out_specs=pl.BlockSpec((_S, NCLS, 8, 128), lambda i: (i, 0, 0, 0)),
        out_shape=jax.ShapeDtypeStruct((B, NCLS, 8, 128), jnp.float32),
    )(gv, a2, b2)

    out = o4.reshape(B, NCLS, 4, 2, 128)
    out = jnp.transpose(out, (0, 1, 2, 4, 3))
    return out.reshape(B, NCLS, PAIRS, 2)
